# baseline (speedup 1.0000x reference)
import jax
import jax.numpy as jnp
from jax import lax
from jax.experimental import pallas as pl
from jax.experimental.pallas import tpu as pltpu
from jax.experimental.pallas import tpu_sc as plsc

B = 4096
L = 50
D = 32
NUM_CORES = 2
NUM_SUBCORES = 16
NW = NUM_CORES * NUM_SUBCORES
BPW = B // NW
SC_ROWS = 32
NSC = BPW // SC_ROWS
GCH = 64
GPS = SC_ROWS * L // GCH
IDX_ROWS = BPW * L // GCH
HALF = 16


def _sc_body(table_ref, idx_ref, out_ref, idx_v, rows0, rows1, out_v,
             sem0, sem1):
    wid = lax.axis_index("s") * NUM_CORES + lax.axis_index("c")

    pltpu.sync_copy(idx_ref.at[wid], idx_v)

    rows = (rows0, rows1)
    sems = (sem0, sem1)

    def fire(sc):
        buf = rows[sc % 2]
        sem = sems[sc % 2]
        handles = []
        for m in range(GPS):
            j = sc * GPS + m
            h = pltpu.async_copy(
                table_ref.at[idx_v.at[j]],
                buf.at[pl.ds(m * GCH, GCH)],
                sem,
            )
            handles.append(h)
        return handles

    inv_l = jnp.float32(1.0 / L)

    def reduce_chunk(sc):
        buf = rows[sc % 2]

        def body(b, carry):
            r0 = b * L
            for h in range(2):
                col = pl.ds(h * HALF, HALF)
                parts = [buf[r0 + k, col] for k in range(4)]
                for l in range(4, L):
                    parts[l % 4] = parts[l % 4] + buf[r0 + l, col]
                s = (parts[0] + parts[1]) + (parts[2] + parts[3])
                out_v[b, col] = s * inv_l
            return carry

        lax.fori_loop(0, SC_ROWS, body, 0)
        row0 = wid * BPW + sc * SC_ROWS
        pltpu.sync_copy(out_v, out_ref.at[pl.ds(row0, SC_ROWS)])

    pending = fire(0)
    for sc in range(NSC):
        nxt = fire(sc + 1) if sc + 1 < NSC else []
        for h in pending:
            h.wait()
        pending = nxt
        reduce_chunk(sc)


def kernel(inputs, table):
    idx2 = inputs.reshape(NW, IDX_ROWS, GCH)
    mesh = plsc.VectorSubcoreMesh(core_axis_name="c", subcore_axis_name="s")
    k = pl.kernel(
        _sc_body,
        out_type=jax.ShapeDtypeStruct((B, D), jnp.float32),
        mesh=mesh,
        scratch_types=[
            pltpu.VMEM((IDX_ROWS, GCH), jnp.int32),
            pltpu.VMEM((SC_ROWS * L, D), jnp.float32),
            pltpu.VMEM((SC_ROWS * L, D), jnp.float32),
            pltpu.VMEM((SC_ROWS, D), jnp.float32),
            pltpu.SemaphoreType.DMA,
            pltpu.SemaphoreType.DMA,
        ],
        compiler_params=pltpu.CompilerParams(use_tc_tiling_on_sc=False),
    )
    return k(table, idx2)
